# baseline (speedup 1.0000x reference)
import functools

import jax
import jax.numpy as jnp
from jax import lax
from jax.experimental import pallas as pl
from jax.experimental.pallas import tpu as pltpu
from jax.experimental.pallas import tpu_sc as plsc

BATCH = 16384
FEAT = 128
NUM_CORES = 2
NUM_SUBCORES = 16
NUM_WORKERS = NUM_CORES * NUM_SUBCORES
ROWS_PER_WORKER = BATCH // NUM_WORKERS
CHUNK = 128
NUM_CHUNKS = ROWS_PER_WORKER // CHUNK
LANES = 16
VECS_PER_ROW = FEAT // LANES

_mesh = plsc.VectorSubcoreMesh(
    core_axis_name="c", subcore_axis_name="s", num_cores=NUM_CORES
)


@functools.partial(
    pl.kernel,
    out_type=jax.ShapeDtypeStruct((NUM_WORKERS, LANES), jnp.float32),
    mesh=_mesh,
    scratch_types=[
        pltpu.VMEM((ROWS_PER_WORKER,), jnp.int32),
        pltpu.VMEM((NUM_CHUNKS, CHUNK, FEAT), jnp.float32),
        pltpu.VMEM((2, CHUNK, FEAT), jnp.float32),
        pltpu.VMEM((LANES,), jnp.float32),
        pltpu.SemaphoreType.DMA,
        pltpu.SemaphoreType.DMA,
        pltpu.SemaphoreType.DMA,
        pltpu.SemaphoreType.DMA,
        pltpu.SemaphoreType.DMA,
        pltpu.SemaphoreType.DMA,
    ],
)
def _center_loss_partials(
    features_hbm,
    labels_hbm,
    centers_hbm,
    out_hbm,
    lab_v,
    f_v,
    c_v,
    acc_v,
    fsem0,
    fsem1,
    fsem2,
    fsem3,
    gsem0,
    gsem1,
):
    wid = lax.axis_index("s") * NUM_CORES + lax.axis_index("c")
    base = wid * ROWS_PER_WORKER

    fsems = (fsem0, fsem1, fsem2, fsem3)
    gsems = (gsem0, gsem1)

    fcps = []
    for k in range(NUM_CHUNKS):
        fcp = pltpu.make_async_copy(
            features_hbm.at[pl.ds(base + k * CHUNK, CHUNK), :],
            f_v.at[k],
            fsems[k],
        )
        fcp.start()
        fcps.append(fcp)

    pltpu.sync_copy(labels_hbm.at[pl.ds(base, ROWS_PER_WORKER)], lab_v)

    def start_gather(k):
        gcp = pltpu.make_async_copy(
            centers_hbm.at[lab_v.at[pl.ds(k * CHUNK, CHUNK)]],
            c_v.at[k % 2],
            gsems[k % 2],
        )
        gcp.start()
        return gcp

    inflight = start_gather(0)
    acc = jnp.zeros((LANES,), jnp.float32)

    for k in range(NUM_CHUNKS):
        gcp = inflight
        if k + 1 < NUM_CHUNKS:
            inflight = start_gather(k + 1)
        fcps[k].wait()
        gcp.wait()

        fb = f_v.at[k]
        cb = c_v.at[k % 2]

        def row_body(r, acc):
            for j in range(VECS_PER_ROW):
                d = fb[r, pl.ds(j * LANES, LANES)] - cb[r, pl.ds(j * LANES, LANES)]
                acc = acc + d * d
            return acc

        acc = lax.fori_loop(0, CHUNK, row_body, acc)

    acc_v[...] = acc
    pltpu.sync_copy(acc_v, out_hbm.at[wid])


def kernel(features, labels, centers):
    labels = labels.astype(jnp.int32)
    partials = _center_loss_partials(features, labels, centers)
    return jnp.sum(partials) / 2.0 / BATCH


# speedup vs baseline: 1.0146x; 1.0146x over previous
import functools

import jax
import jax.numpy as jnp
from jax import lax
from jax.experimental import pallas as pl
from jax.experimental.pallas import tpu as pltpu
from jax.experimental.pallas import tpu_sc as plsc

BATCH = 16384
FEAT = 128
NUM_CORES = 2
NUM_SUBCORES = 16
NUM_WORKERS = NUM_CORES * NUM_SUBCORES
ROWS_PER_WORKER = BATCH // NUM_WORKERS
CHUNK = 128
NUM_CHUNKS = ROWS_PER_WORKER // CHUNK
LANES = 16
VECS_PER_ROW = FEAT // LANES

_mesh = plsc.VectorSubcoreMesh(
    core_axis_name="c", subcore_axis_name="s", num_cores=NUM_CORES
)


@functools.partial(
    pl.kernel,
    out_type=jax.ShapeDtypeStruct((NUM_WORKERS, LANES), jnp.float32),
    mesh=_mesh,
    scratch_types=[
        pltpu.VMEM((ROWS_PER_WORKER,), jnp.int32),
        pltpu.VMEM((2, CHUNK, FEAT), jnp.float32),
        pltpu.VMEM((2, CHUNK, FEAT), jnp.float32),
        pltpu.VMEM((LANES,), jnp.float32),
        pltpu.SemaphoreType.DMA,
        pltpu.SemaphoreType.DMA,
        pltpu.SemaphoreType.DMA,
        pltpu.SemaphoreType.DMA,
    ],
)
def _center_loss_partials(
    features_hbm,
    labels_hbm,
    centers_hbm,
    out_hbm,
    lab_v,
    f_v,
    c_v,
    acc_v,
    fsem0,
    fsem1,
    gsem0,
    gsem1,
):
    wid = lax.axis_index("s") * NUM_CORES + lax.axis_index("c")
    base = wid * ROWS_PER_WORKER

    pltpu.sync_copy(labels_hbm.at[pl.ds(base, ROWS_PER_WORKER)], lab_v)

    fsems = (fsem0, fsem1)
    gsems = (gsem0, gsem1)

    def start_chunk(k):
        buf = k % 2
        row0 = base + k * CHUNK
        fcp = pltpu.make_async_copy(
            features_hbm.at[pl.ds(row0, CHUNK), :], f_v.at[buf], fsems[buf]
        )
        gcp = pltpu.make_async_copy(
            centers_hbm.at[lab_v.at[pl.ds(k * CHUNK, CHUNK)]],
            c_v.at[buf],
            gsems[buf],
        )
        fcp.start()
        gcp.start()
        return fcp, gcp

    inflight = start_chunk(0)
    acc = jnp.zeros((LANES,), jnp.float32)

    for k in range(NUM_CHUNKS):
        fcp, gcp = inflight
        if k + 1 < NUM_CHUNKS:
            inflight = start_chunk(k + 1)
        fcp.wait()
        gcp.wait()

        buf = k % 2
        fb = f_v.at[buf]
        cb = c_v.at[buf]

        def row_body(r, acc):
            for j in range(VECS_PER_ROW):
                d = fb[r, pl.ds(j * LANES, LANES)] - cb[r, pl.ds(j * LANES, LANES)]
                acc = acc + d * d
            return acc

        acc = lax.fori_loop(0, CHUNK, row_body, acc)

    acc_v[...] = acc
    pltpu.sync_copy(acc_v, out_hbm.at[wid])


def kernel(features, labels, centers):
    labels = labels.astype(jnp.int32)
    partials = _center_loss_partials(features, labels, centers)
    return jnp.sum(partials) / 2.0 / BATCH
